# Initial kernel scaffold; baseline (speedup 1.0000x reference)
import functools

import jax
import jax.numpy as jnp
from jax import lax
from jax.experimental import pallas as pl
from jax.experimental.pallas import tpu as pltpu
from jax.experimental.pallas import tpu_sc as plsc

B, L, V, D = 4096, 200, 1000000, 64

NC, NS, LANES = 2, 16, 16
NW = NC * NS
RPW = B // NW
LH = L // 2

_mesh = plsc.VectorSubcoreMesh(core_axis_name="c", subcore_axis_name="s")


@functools.partial(
    pl.kernel,
    out_type=jax.ShapeDtypeStruct((B, D), jnp.float32),
    mesh=_mesh,
    scratch_types=[
        pltpu.VMEM((RPW, L), jnp.int32),
        pltpu.VMEM((RPW,), jnp.int32),
        pltpu.VMEM((2, L, D), jnp.float32),
        pltpu.VMEM((RPW, D), jnp.float32),
        pltpu.SemaphoreType.DMA,
        pltpu.SemaphoreType.DMA,
    ],
)
def _bag_sum_kernel(batch_hbm, lengths_hbm, table_hbm, out_hbm,
                    idx_v, len_v, rows_v, sums_v, sem0, sem1):
    wid = lax.axis_index("s") * NC + lax.axis_index("c")
    base = wid * RPW

    pltpu.sync_copy(batch_hbm.at[pl.ds(base, RPW)], idx_v)
    pltpu.sync_copy(lengths_hbm.at[pl.ds(base, RPW)], len_v)

    def issue(r, buf, sem):
        pltpu.async_copy(table_hbm.at[idx_v.at[r, pl.ds(0, LH)]],
                         rows_v.at[buf, pl.ds(0, LH)], sem)
        pltpu.async_copy(table_hbm.at[idx_v.at[r, pl.ds(LH, LH)]],
                         rows_v.at[buf, pl.ds(LH, LH)], sem)

    def wait(r, buf, sem):
        pltpu.make_async_copy(table_hbm.at[idx_v.at[r, pl.ds(0, LH)]],
                              rows_v.at[buf, pl.ds(0, LH)], sem).wait()
        pltpu.make_async_copy(table_hbm.at[idx_v.at[r, pl.ds(LH, LH)]],
                              rows_v.at[buf, pl.ds(LH, LH)], sem).wait()

    def accumulate(r, buf):
        grp = (r // LANES) * LANES
        lane = r - grp
        lv = len_v[pl.ds(grp, LANES)]
        lane_ids = lax.iota(jnp.int32, LANES)
        len_s = jnp.sum(jnp.where(lane_ids == lane, lv, 0))

        ngroups = (len_s + (LANES - 1)) // LANES

        def group_body(g, accs):
            gbase = g * LANES
            new = []
            for k in range(D // LANES):
                acc = accs[k]
                for j in range(LANES):
                    m = ((gbase + j) < len_s).astype(jnp.float32)
                    acc = acc + rows_v[buf, gbase + j, pl.ds(k * LANES, LANES)] * m
                new.append(acc)
            return tuple(new)

        zero = jnp.zeros((LANES,), jnp.float32)
        accs = lax.fori_loop(0, ngroups, group_body,
                             (zero,) * (D // LANES))
        for k in range(D // LANES):
            sums_v[r, pl.ds(k * LANES, LANES)] = accs[k]

    issue(0, 0, sem0)

    def row_pair(r2, _):
        r = r2 * 2
        issue(r + 1, 1, sem1)
        wait(r, 0, sem0)
        accumulate(r, 0)

        @pl.when(r + 2 < RPW)
        def _():
            issue(r + 2, 0, sem0)

        wait(r + 1, 1, sem1)
        accumulate(r + 1, 1)
        return 0

    lax.fori_loop(0, RPW // 2, row_pair, 0)
    pltpu.sync_copy(sums_v, out_hbm.at[pl.ds(base, RPW)])


def _mlp_body(sums_ref, inv_len_ref, W1_ref, b1_ref, W2_ref, b2_ref, out_ref):
    bag = sums_ref[...] * inv_len_ref[...]
    h = jnp.maximum(
        jnp.dot(bag, W1_ref[...], preferred_element_type=jnp.float32)
        + b1_ref[...], 0.0)
    out_ref[...] = (
        jnp.dot(h, W2_ref[...], preferred_element_type=jnp.float32)
        + b2_ref[...])


def kernel(batch, lengths, table, W1, b1, W2, b2):
    batch = batch.astype(jnp.int32)
    lengths = lengths.astype(jnp.int32)

    sums = _bag_sum_kernel(batch, lengths, table)

    inv_len = (1.0 / jnp.maximum(lengths, 1).astype(jnp.float32))[:, None]
    W2p = jnp.zeros((D // 2, 128), jnp.float32).at[:, :2].set(W2)
    b2p = jnp.zeros((1, 128), jnp.float32).at[0, :2].set(b2)

    logits_p = pl.pallas_call(
        _mlp_body,
        out_shape=jax.ShapeDtypeStruct((B, 128), jnp.float32),
    )(sums, inv_len, W1, b1[None, :], W2p, b2p)
    return logits_p[:, :2]


# speedup vs baseline: 1.0568x; 1.0568x over previous
import functools

import jax
import jax.numpy as jnp
from jax import lax
from jax.experimental import pallas as pl
from jax.experimental.pallas import tpu as pltpu
from jax.experimental.pallas import tpu_sc as plsc

B, L, V, D = 4096, 200, 1000000, 64

NC, NS, LANES = 2, 16, 16
NW = NC * NS
RPW = B // NW
LH1, LH2 = 128, 72

_mesh = plsc.VectorSubcoreMesh(core_axis_name="c", subcore_axis_name="s")


@functools.partial(
    pl.kernel,
    out_type=jax.ShapeDtypeStruct((B, D), jnp.float32),
    mesh=_mesh,
    compiler_params=pltpu.CompilerParams(use_tc_tiling_on_sc=False,
                                         needs_layout_passes=False),
    scratch_types=[
        pltpu.VMEM((RPW, L), jnp.int32),
        pltpu.VMEM((RPW,), jnp.int32),
        pltpu.VMEM((2, L, D), jnp.float32),
        pltpu.VMEM((RPW, D), jnp.float32),
        pltpu.SemaphoreType.DMA,
        pltpu.SemaphoreType.DMA,
    ],
)
def _bag_sum_kernel(batch_hbm, lengths_hbm, table_hbm, out_hbm,
                    idx_v, len_v, rows_v, sums_v, sem0, sem1):
    wid = lax.axis_index("s") * NC + lax.axis_index("c")
    base = wid * RPW

    pltpu.sync_copy(batch_hbm.at[pl.ds(base, RPW)], idx_v)
    pltpu.sync_copy(lengths_hbm.at[pl.ds(base, RPW)], len_v)

    def issue(r, buf, sem):
        pltpu.async_copy(table_hbm.at[idx_v.at[r, pl.ds(0, LH1)]],
                         rows_v.at[buf, pl.ds(0, LH1)], sem)
        pltpu.async_copy(table_hbm.at[idx_v.at[r, pl.ds(LH1, LH2)]],
                         rows_v.at[buf, pl.ds(LH1, LH2)], sem)

    def wait(r, buf, sem):
        pltpu.make_async_copy(table_hbm.at[idx_v.at[r, pl.ds(0, LH1)]],
                              rows_v.at[buf, pl.ds(0, LH1)], sem).wait()
        pltpu.make_async_copy(table_hbm.at[idx_v.at[r, pl.ds(LH1, LH2)]],
                              rows_v.at[buf, pl.ds(LH1, LH2)], sem).wait()

    def accumulate(r, buf):
        grp = (r // LANES) * LANES
        lane = r - grp
        lv = len_v[pl.ds(grp, LANES)]
        lane_ids = lax.iota(jnp.int32, LANES)
        len_s = jnp.sum(jnp.where(lane_ids == lane, lv, 0))

        ngroups = (len_s + (LANES - 1)) // LANES

        def group_body(g, accs):
            gbase = g * LANES
            new = []
            for k in range(D // LANES):
                acc = accs[k]
                for j in range(LANES):
                    m = ((gbase + j) < len_s).astype(jnp.float32)
                    acc = acc + rows_v[buf, gbase + j, pl.ds(k * LANES, LANES)] * m
                new.append(acc)
            return tuple(new)

        zero = jnp.zeros((LANES,), jnp.float32)
        accs = lax.fori_loop(0, ngroups, group_body,
                             (zero,) * (D // LANES))
        for k in range(D // LANES):
            sums_v[r, pl.ds(k * LANES, LANES)] = accs[k]

    issue(0, 0, sem0)

    def row_pair(r2, _):
        r = r2 * 2
        issue(r + 1, 1, sem1)
        wait(r, 0, sem0)
        accumulate(r, 0)

        @pl.when(r + 2 < RPW)
        def _():
            issue(r + 2, 0, sem0)

        wait(r + 1, 1, sem1)
        accumulate(r + 1, 1)
        return 0

    lax.fori_loop(0, RPW // 2, row_pair, 0)
    pltpu.sync_copy(sums_v, out_hbm.at[pl.ds(base, RPW)])


def _mlp_body(sums_ref, inv_len_ref, W1_ref, b1_ref, W2_ref, b2_ref, out_ref):
    bag = sums_ref[...] * inv_len_ref[...]
    h = jnp.maximum(
        jnp.dot(bag, W1_ref[...], preferred_element_type=jnp.float32)
        + b1_ref[...], 0.0)
    out_ref[...] = (
        jnp.dot(h, W2_ref[...], preferred_element_type=jnp.float32)
        + b2_ref[...])


def kernel(batch, lengths, table, W1, b1, W2, b2):
    batch = batch.astype(jnp.int32)
    lengths = lengths.astype(jnp.int32)

    sums = _bag_sum_kernel(batch, lengths, table)

    inv_len = (1.0 / jnp.maximum(lengths, 1).astype(jnp.float32))[:, None]
    W2p = jnp.zeros((D // 2, 128), jnp.float32).at[:, :2].set(W2)
    b2p = jnp.zeros((1, 128), jnp.float32).at[0, :2].set(b2)

    logits_p = pl.pallas_call(
        _mlp_body,
        out_shape=jax.ShapeDtypeStruct((B, 128), jnp.float32),
    )(sums, inv_len, W1, b1[None, :], W2p, b2p)
    return logits_p[:, :2]
